# baseline (speedup 1.0000x reference)
import functools

import jax
import jax.numpy as jnp
from jax import lax
from jax.experimental import pallas as pl
from jax.experimental.pallas import tpu as pltpu
from jax.experimental.pallas import tpu_sc as plsc

_VOCAB = 1000000
_MOD = _VOCAB - 1
_FOLD = 48577
_D = 64
_DP = 128
_B = 4096
_S = 200
_NCHUNK = 13

_NC, _NS = 2, 16
_NW = _NC * _NS
_ROWS_PER_W = _B // _NW
_TOKS_PER_W = _ROWS_PER_W * _S

_G = 1
_GIDX = _G * _S
_NG = _ROWS_PER_W // _G
_GCHUNKS = [(0, 104), (104, 96)]

_DCH = 200
_NDCH = _VOCAB // _DCH


def _fold_mod(x):
    for _ in range(3):
        x = (x >> 20) * jnp.int32(_FOLD) + (x & jnp.int32(0xFFFFF))
    x = jnp.where(x < 0, x + jnp.int32(_MOD), x)
    x = jnp.where(x >= jnp.int32(_MOD), x - jnp.int32(_MOD), x)
    return x


@functools.partial(
    pl.kernel,
    out_type=jax.ShapeDtypeStruct((_B, _S, _D), jnp.float32),
    mesh=plsc.VectorSubcoreMesh(core_axis_name="c", subcore_axis_name="s"),
    scratch_types=[
        pltpu.HBM((_VOCAB, _DP), jnp.float32),
        pltpu.VMEM((16 + _TOKS_PER_W + 8,), jnp.int32),
        pltpu.VMEM((_GIDX + 8,), jnp.int32),
        pltpu.VMEM((_GIDX + 8,), jnp.int32),
        pltpu.VMEM((_GIDX, _DP), jnp.float32),
        pltpu.VMEM((_GIDX, _DP), jnp.float32),
        pltpu.VMEM((_DCH, _D), jnp.float32),
        pltpu.VMEM((_DCH, _D), jnp.float32),
        pltpu.VMEM((128,), jnp.float32),
        pltpu.SemaphoreType.DMA,
        pltpu.SemaphoreType.DMA,
        pltpu.SemaphoreType.DMA,
        pltpu.SemaphoreType.DMA,
    ],
)
def _sc_embed(tok_hbm, table_hbm, scale_hbm, out_hbm,
              tbl_pad, tok_v, idx_v0, idx_v1, rows_v0, rows_v1,
              cmp_v0, cmp_v1, scale_v, gsem0, gsem1, wsem0, wsem1):
    sid = lax.axis_index("s")
    wid = sid * _NC + lax.axis_index("c")
    base_row = wid * _ROWS_PER_W

    idx_bufs = (idx_v0, idx_v1)
    row_bufs = (rows_v0, rows_v1)
    cmp_bufs = (cmp_v0, cmp_v1)
    gsems = (gsem0, gsem1)
    wsems = (wsem0, wsem1)

    pltpu.sync_copy(scale_hbm, scale_v)
    sval = scale_v[pl.ds(0, 16)]
    tok_v[pl.ds(0, 16)] = jnp.zeros((16,), jnp.int32)
    pltpu.sync_copy(tok_hbm.at[pl.ds(base_row * _S, _TOKS_PER_W)],
                    tok_v.at[pl.ds(8, _TOKS_PER_W)])

    def dstep(i2, carry):
        for half in range(2):
            k = 2 * i2 + half
            ci = sid + _NS * k
            c0 = ci * _DCH

            def wait_out_km2():
                pltpu.make_async_copy(
                    row_bufs[half],
                    tbl_pad.at[pl.ds(c0 - 2 * _NS * _DCH, _DCH)],
                    wsems[half]).wait()
            pl.when(jnp.logical_and(k >= 2, ci - 2 * _NS < _NDCH))(wait_out_km2)

            def fire_in_k():
                pltpu.async_copy(table_hbm.at[pl.ds(c0, _DCH)],
                                 cmp_bufs[half], gsems[half])
            pl.when(ci < _NDCH)(fire_in_k)

            def drain_km1():
                pltpu.make_async_copy(
                    table_hbm.at[pl.ds(c0 - _NS * _DCH, _DCH)],
                    cmp_bufs[1 - half], gsems[1 - half]).wait()

                def interleave(i, carry):
                    for kk in range(_D // 16):
                        row_bufs[1 - half][i, pl.ds(16 * kk, 16)] = (
                            cmp_bufs[1 - half][i, pl.ds(16 * kk, 16)])
                    return carry
                lax.fori_loop(0, _DCH, interleave, 0, unroll=4)
                pltpu.async_copy(row_bufs[1 - half],
                                 tbl_pad.at[pl.ds(c0 - _NS * _DCH, _DCH)],
                                 wsems[1 - half])
            pl.when(jnp.logical_and(k >= 1, ci - _NS < _NDCH))(drain_km1)
        return carry

    lax.fori_loop(0, (_NDCH // _NS) // 2 + 2, dstep, 0)
    plsc.subcore_barrier()

    def hash_group(g, idx_ref):
        soff = g * _GIDX
        for j in range(_NCHUNK):
            a = tok_v[pl.ds(soff + 16 * j + 8, 16)]
            b = tok_v[pl.ds(soff + 16 * j + 7, 16)]
            c = tok_v[pl.ds(soff + 16 * j + 6, 16)]
            x2 = (a * jnp.int32(36313)) ^ (b * jnp.int32(27191))
            x3 = x2 ^ (c * jnp.int32(51647))
            if j == 0:
                lane = lax.iota(jnp.int32, 16)
                h = jnp.where(lane == 0, jnp.int32(_MOD),
                              jnp.where(lane == 1, _fold_mod(x2),
                                        _fold_mod(x3)))
            else:
                h = _fold_mod(x3)
            idx_ref[pl.ds(16 * j, 16)] = h

    def fire_gathers(idx_ref, rows_ref, sem):
        for o, n in _GCHUNKS:
            pltpu.async_copy(tbl_pad.at[idx_ref.at[pl.ds(o, n)]],
                             rows_ref.at[pl.ds(o, n)], sem)

    def wait_gathers(idx_ref, rows_ref, sem):
        for o, n in _GCHUNKS:
            pltpu.make_async_copy(tbl_pad.at[idx_ref.at[pl.ds(o, n)]],
                                  rows_ref.at[pl.ds(o, n)], sem).wait()

    def scale_compact(rows_ref, cmp_ref):
        def body(i, carry):
            for k in range(_D // 16):
                cmp_ref[i, pl.ds(16 * k, 16)] = (
                    rows_ref[i, pl.ds(16 * k, 16)] * sval)
            return carry
        lax.fori_loop(0, _GIDX, body, 0, unroll=4)

    def fire_writeback(g, cmp_ref, sem):
        pltpu.async_copy(cmp_ref, out_hbm.at[base_row + g], sem)

    def wait_writeback(g, cmp_ref, sem):
        pltpu.make_async_copy(cmp_ref, out_hbm.at[base_row + g], sem).wait()

    hash_group(jnp.int32(0), idx_bufs[0])
    fire_gathers(idx_bufs[0], row_bufs[0], gsems[0])

    def body(i2, carry):
        for half in range(2):
            g = 2 * i2 + half
            ng = g + 1
            other = 1 - half

            def launch_next():
                hash_group(ng, idx_bufs[other])
                fire_gathers(idx_bufs[other], row_bufs[other], gsems[other])

            if half == 0:
                launch_next()
            else:
                pl.when(ng < _NG)(launch_next)

            wait_gathers(idx_bufs[half], row_bufs[half], gsems[half])

            def free_cmp():
                wait_writeback(g - 2, cmp_bufs[half], wsems[half])
            pl.when(g >= 2)(free_cmp)
            scale_compact(row_bufs[half], cmp_bufs[half])
            fire_writeback(g, cmp_bufs[half], wsems[half])
        return carry

    lax.fori_loop(0, _NG // 2, body, 0)
    wait_writeback(_NG - 2, cmp_bufs[0], wsems[0])
    wait_writeback(_NG - 1, cmp_bufs[1], wsems[1])


def kernel(token_ids, embed_table, scale):
    scale_vec = jnp.full((128,), scale, dtype=jnp.float32)
    return _sc_embed(token_ids.reshape(-1), embed_table, scale_vec)


# speedup vs baseline: 1.7014x; 1.7014x over previous
import functools

import jax
import jax.numpy as jnp
from jax import lax
from jax.experimental import pallas as pl
from jax.experimental.pallas import tpu as pltpu
from jax.experimental.pallas import tpu_sc as plsc

_VOCAB = 1000000
_MOD = _VOCAB - 1
_FOLD = 48577
_D = 64
_DP = 128
_B = 4096
_S = 200
_NCHUNK = 13

_NC, _NS = 2, 16
_NW = _NC * _NS
_ROWS_PER_W = _B // _NW
_TOKS_PER_W = _ROWS_PER_W * _S

_G = 1
_GIDX = _G * _S
_NG = _ROWS_PER_W // _G
_GCHUNKS = [(0, 104), (104, 96)]


def _fold_mod(x):
    for _ in range(3):
        x = (x >> 20) * jnp.int32(_FOLD) + (x & jnp.int32(0xFFFFF))
    x = jnp.where(x < 0, x + jnp.int32(_MOD), x)
    x = jnp.where(x >= jnp.int32(_MOD), x - jnp.int32(_MOD), x)
    return x


@functools.partial(
    pl.kernel,
    out_type=jax.ShapeDtypeStruct((_B, _S, _D), jnp.float32),
    mesh=plsc.VectorSubcoreMesh(core_axis_name="c", subcore_axis_name="s"),
    scratch_types=[
        pltpu.VMEM((16 + _TOKS_PER_W + 8,), jnp.int32),
        pltpu.VMEM((_GIDX + 8,), jnp.int32),
        pltpu.VMEM((_GIDX + 8,), jnp.int32),
        pltpu.VMEM((_GIDX, _DP), jnp.float32),
        pltpu.VMEM((_GIDX, _DP), jnp.float32),
        pltpu.VMEM((_GIDX, _D), jnp.float32),
        pltpu.VMEM((_GIDX, _D), jnp.float32),
        pltpu.VMEM((128,), jnp.float32),
        pltpu.SemaphoreType.DMA,
        pltpu.SemaphoreType.DMA,
        pltpu.SemaphoreType.DMA,
        pltpu.SemaphoreType.DMA,
    ],
)
def _sc_embed(tok_hbm, table_hbm, scale_hbm, out_hbm,
              tok_v, idx_v0, idx_v1, rows_v0, rows_v1,
              cmp_v0, cmp_v1, scale_v, gsem0, gsem1, wsem0, wsem1):
    sid = lax.axis_index("s")
    wid = sid * _NC + lax.axis_index("c")
    base_row = wid * _ROWS_PER_W

    idx_bufs = (idx_v0, idx_v1)
    row_bufs = (rows_v0, rows_v1)
    cmp_bufs = (cmp_v0, cmp_v1)
    gsems = (gsem0, gsem1)
    wsems = (wsem0, wsem1)

    pltpu.sync_copy(scale_hbm, scale_v)
    sval = scale_v[pl.ds(0, 16)]
    tok_v[pl.ds(0, 16)] = jnp.zeros((16,), jnp.int32)
    pltpu.sync_copy(tok_hbm.at[pl.ds(base_row * _S, _TOKS_PER_W)],
                    tok_v.at[pl.ds(8, _TOKS_PER_W)])

    def hash_group(g, idx_ref):
        soff = g * _GIDX
        for j in range(_NCHUNK):
            a = tok_v[pl.ds(soff + 16 * j + 8, 16)]
            b = tok_v[pl.ds(soff + 16 * j + 7, 16)]
            c = tok_v[pl.ds(soff + 16 * j + 6, 16)]
            x2 = (a * jnp.int32(36313)) ^ (b * jnp.int32(27191))
            x3 = x2 ^ (c * jnp.int32(51647))
            if j == 0:
                lane = lax.iota(jnp.int32, 16)
                h = jnp.where(lane == 0, jnp.int32(_MOD),
                              jnp.where(lane == 1, _fold_mod(x2),
                                        _fold_mod(x3)))
            else:
                h = _fold_mod(x3)
            idx_ref[pl.ds(16 * j, 16)] = h

    def fire_gathers(idx_ref, rows_ref, sem):
        for o, n in _GCHUNKS:
            pltpu.async_copy(table_hbm.at[idx_ref.at[pl.ds(o, n)]],
                             rows_ref.at[pl.ds(o, n)], sem)

    def wait_gathers(idx_ref, rows_ref, sem):
        for o, n in _GCHUNKS:
            pltpu.make_async_copy(table_hbm.at[idx_ref.at[pl.ds(o, n)]],
                                  rows_ref.at[pl.ds(o, n)], sem).wait()

    def scale_compact(rows_ref, cmp_ref):
        def body(i, carry):
            for k in range(_D // 16):
                cmp_ref[i, pl.ds(16 * k, 16)] = (
                    rows_ref[i, pl.ds(16 * k, 16)] * sval)
            return carry
        lax.fori_loop(0, _GIDX, body, 0, unroll=4)

    def fire_writeback(g, cmp_ref, sem):
        pltpu.async_copy(cmp_ref, out_hbm.at[base_row + g], sem)

    def wait_writeback(g, cmp_ref, sem):
        pltpu.make_async_copy(cmp_ref, out_hbm.at[base_row + g], sem).wait()

    hash_group(jnp.int32(0), idx_bufs[0])
    fire_gathers(idx_bufs[0], row_bufs[0], gsems[0])

    def body(i2, carry):
        for half in range(2):
            g = 2 * i2 + half
            ng = g + 1
            other = 1 - half

            def launch_next():
                hash_group(ng, idx_bufs[other])
                fire_gathers(idx_bufs[other], row_bufs[other], gsems[other])

            if half == 0:
                launch_next()
            else:
                pl.when(ng < _NG)(launch_next)

            wait_gathers(idx_bufs[half], row_bufs[half], gsems[half])

            def free_cmp():
                wait_writeback(g - 2, cmp_bufs[half], wsems[half])
            pl.when(g >= 2)(free_cmp)
            scale_compact(row_bufs[half], cmp_bufs[half])
            fire_writeback(g, cmp_bufs[half], wsems[half])
        return carry

    lax.fori_loop(0, _NG // 2, body, 0)
    wait_writeback(_NG - 2, cmp_bufs[0], wsems[0])
    wait_writeback(_NG - 1, cmp_bufs[1], wsems[1])


def kernel(token_ids, embed_table, scale):
    table_wide = jnp.pad(embed_table, ((0, 0), (0, _DP - _D)))
    scale_vec = jnp.full((128,), scale, dtype=jnp.float32)
    return _sc_embed(token_ids.reshape(-1), table_wide, scale_vec)


# speedup vs baseline: 1.7340x; 1.0192x over previous
import functools

import jax
import jax.numpy as jnp
from jax import lax
from jax.experimental import pallas as pl
from jax.experimental.pallas import tpu as pltpu
from jax.experimental.pallas import tpu_sc as plsc

_VOCAB = 1000000
_MOD = _VOCAB - 1
_FOLD = 48577
_D = 64
_B = 4096
_S = 200
_NCHUNK = 13

_NC, _NS = 2, 16
_NW = _NC * _NS
_ROWS_PER_W = _B // _NW

_G = 4
_GIDX = _G * _S
_NG = _ROWS_PER_W // _G
_GCHUNKS = [(o, min(128, _GIDX - o)) for o in range(0, _GIDX, 128)]


def _fold_mod(x):
    for _ in range(3):
        x = (x >> 20) * jnp.int32(_FOLD) + (x & jnp.int32(0xFFFFF))
    x = jnp.where(x < 0, x + jnp.int32(_MOD), x)
    x = jnp.where(x >= jnp.int32(_MOD), x - jnp.int32(_MOD), x)
    return x


@functools.partial(
    pl.kernel,
    out_type=jax.ShapeDtypeStruct((_B, _S, _D), jnp.float32),
    mesh=plsc.VectorSubcoreMesh(core_axis_name="c", subcore_axis_name="s"),
    compiler_params=pltpu.CompilerParams(use_tc_tiling_on_sc=False),
    scratch_types=[
        pltpu.VMEM((16 + _GIDX,), jnp.int32),
        pltpu.VMEM((_GIDX + 8,), jnp.int32),
        pltpu.VMEM((_GIDX + 8,), jnp.int32),
        pltpu.VMEM((_GIDX, _D), jnp.float32),
        pltpu.VMEM((_GIDX, _D), jnp.float32),
        pltpu.VMEM((16,), jnp.float32),
        pltpu.SemaphoreType.DMA,
        pltpu.SemaphoreType.DMA,
        pltpu.SemaphoreType.DMA,
        pltpu.SemaphoreType.DMA,
    ],
)
def _sc_embed(tok_hbm, table_hbm, scale_hbm, out_hbm,
              tok_v, idx_v0, idx_v1, rows_v0, rows_v1, scale_v,
              gsem0, gsem1, wsem0, wsem1):
    wid = lax.axis_index("s") * _NC + lax.axis_index("c")
    base_row = wid * _ROWS_PER_W
    table2d = table_hbm

    idx_bufs = (idx_v0, idx_v1)
    row_bufs = (rows_v0, rows_v1)
    gsems = (gsem0, gsem1)
    wsems = (wsem0, wsem1)

    pltpu.sync_copy(scale_hbm, scale_v)
    sval = scale_v[...]
    tok_v[pl.ds(0, 16)] = jnp.zeros((16,), jnp.int32)

    def hash_group(g, idx_ref):
        t0 = (base_row + g * _G) * _S
        pltpu.sync_copy(tok_hbm.at[pl.ds(t0, _GIDX)], tok_v.at[pl.ds(8, _GIDX)])

        def row_body(ri, carry):
            toff = ri * _S
            for j in range(_NCHUNK):
                a = tok_v[pl.ds(toff + 16 * j + 8, 16)]
                b = tok_v[pl.ds(toff + 16 * j + 7, 16)]
                c = tok_v[pl.ds(toff + 16 * j + 6, 16)]
                x2 = (a * jnp.int32(36313)) ^ (b * jnp.int32(27191))
                x3 = x2 ^ (c * jnp.int32(51647))
                if j == 0:
                    lane = lax.iota(jnp.int32, 16)
                    h = jnp.where(lane == 0, jnp.int32(_MOD),
                                  jnp.where(lane == 1, _fold_mod(x2),
                                            _fold_mod(x3)))
                else:
                    h = _fold_mod(x3)
                idx_ref[pl.ds(toff + 16 * j, 16)] = h
            return carry

        lax.fori_loop(0, _G, row_body, 0)

    def fire_gathers(idx_ref, rows_ref, sem):
        for o, n in _GCHUNKS:
            pltpu.async_copy(table2d.at[idx_ref.at[pl.ds(o, n)]],
                             rows_ref.at[pl.ds(o, n)], sem)

    def wait_gathers(idx_ref, rows_ref, sem):
        for o, n in _GCHUNKS:
            pltpu.make_async_copy(table2d.at[idx_ref.at[pl.ds(o, n)]],
                                  rows_ref.at[pl.ds(o, n)], sem).wait()

    def scale_rows(rows_ref):
        def body(i, carry):
            for k in range(_D // 16):
                rows_ref[i, pl.ds(16 * k, 16)] = (
                    rows_ref[i, pl.ds(16 * k, 16)] * sval)
            return carry
        lax.fori_loop(0, _GIDX, body, 0, unroll=4)

    def fire_writeback(g, rows_ref, sem):
        r0 = base_row + g * _G
        for ri in range(_G):
            pltpu.async_copy(rows_ref.at[pl.ds(ri * _S, _S)],
                             out_hbm.at[r0 + ri], sem)

    def wait_writeback(g, rows_ref, sem):
        r0 = base_row + g * _G
        for ri in range(_G):
            pltpu.make_async_copy(rows_ref.at[pl.ds(ri * _S, _S)],
                                  out_hbm.at[r0 + ri], sem).wait()

    hash_group(jnp.int32(0), idx_bufs[0])
    fire_gathers(idx_bufs[0], row_bufs[0], gsems[0])

    def body(i2, carry):
        for half in range(2):
            g = 2 * i2 + half
            ng = g + 1
            other = 1 - half

            def launch_next():
                hash_group(ng, idx_bufs[other])

                def free_rows():
                    wait_writeback(ng - 2, row_bufs[other], wsems[other])
                if half == 0:
                    pl.when(ng >= 2)(free_rows)
                else:
                    free_rows()
                fire_gathers(idx_bufs[other], row_bufs[other], gsems[other])

            if half == 0:
                launch_next()
            else:
                pl.when(ng < _NG)(launch_next)

            wait_gathers(idx_bufs[half], row_bufs[half], gsems[half])
            scale_rows(row_bufs[half])
            fire_writeback(g, row_bufs[half], wsems[half])
        return carry

    lax.fori_loop(0, _NG // 2, body, 0)
    wait_writeback(_NG - 2, row_bufs[0], wsems[0])
    wait_writeback(_NG - 1, row_bufs[1], wsems[1])


def kernel(token_ids, embed_table, scale):
    scale_vec = jnp.full((16,), scale, dtype=jnp.float32)
    return _sc_embed(token_ids.reshape(-1), embed_table, scale_vec)
